# Initial kernel scaffold; baseline (speedup 1.0000x reference)
import jax
import jax.numpy as jnp
from jax.experimental import pallas as pl


def kernel(x, edge_index, weight, bias):
    raise NotImplementedError("write your pallas kernel here")


# speedup vs baseline: 12.7082x; 12.7082x over previous
import functools

import jax
import jax.numpy as jnp
from jax import lax
from jax.experimental import pallas as pl
from jax.experimental.pallas import tpu as pltpu
from jax.experimental.pallas import tpu_sc as plsc

N = 10000
E = 320000
F = 144
NC = 2
NS = 16
NW = NC * NS
L = 16
EPW = E // NW
CH = 80
NCHUNK = EPW // CH
NPAD = N + L
RPT = NPAD // NS
WPT = N // NS

_mesh = plsc.VectorSubcoreMesh(core_axis_name="c", subcore_axis_name="s")
_sc_params = pltpu.CompilerParams(needs_layout_passes=False,
                                  use_tc_tiling_on_sc=False)


@functools.partial(
    pl.kernel,
    out_type=[jax.ShapeDtypeStruct((NW, N), jnp.float32),
              jax.ShapeDtypeStruct((E,), jnp.int32)],
    mesh=_mesh,
    compiler_params=_sc_params,
    scratch_types=[pltpu.VMEM((EPW,), jnp.int32),
                   pltpu.VMEM((EPW,), jnp.int32),
                   pltpu.VMEM((EPW,), jnp.int32),
                   pltpu.VMEM((NPAD,), jnp.float32)],
)
def _deg_ridx(row_hbm, col_hbm, pdeg_hbm, ridx_hbm, row_v, col_v, ridx_v, deg_v):
    c = lax.axis_index("c")
    s = lax.axis_index("s")
    wid = s * NC + c
    base = wid * EPW
    pltpu.sync_copy(row_hbm.at[pl.ds(base, EPW)], row_v)
    pltpu.sync_copy(col_hbm.at[pl.ds(base, EPW)], col_v)

    zero = jnp.zeros((L,), jnp.float32)

    def zbody(i, carry):
        deg_v[pl.ds(i * L, L)] = zero
        return carry

    lax.fori_loop(0, NPAD // L, zbody, 0)

    ones = jnp.ones((L,), jnp.float32)

    def ebody(i, carry):
        r = row_v[pl.ds(i * L, L)]
        cc = col_v[pl.ds(i * L, L)]
        r2 = jnp.where(r != cc, r, N)
        plsc.addupdate_scatter(deg_v, [r2], ones)
        ridx_v[pl.ds(i * L, L)] = r2
        return carry

    lax.fori_loop(0, EPW // L, ebody, 0)

    pltpu.sync_copy(deg_v.at[pl.ds(0, N)], pdeg_hbm.at[wid])
    pltpu.sync_copy(ridx_v, ridx_hbm.at[pl.ds(base, EPW)])


@functools.partial(
    pl.kernel,
    out_type=jax.ShapeDtypeStruct((NC, N, F), jnp.float32),
    mesh=_mesh,
    compiler_params=_sc_params,
    scratch_types=[pltpu.VMEM((EPW,), jnp.int32),
                   pltpu.VMEM((EPW,), jnp.int32),
                   pltpu.VMEM((CH,), jnp.int32),
                   pltpu.VMEM((CH,), jnp.int32),
                   pltpu.VMEM((CH, F), jnp.float32),
                   pltpu.VMEM((L, F), jnp.float32),
                   pltpu.VMEM_SHARED((NPAD, F), jnp.float32),
                   pltpu.SemaphoreType.DMA],
)
def _spmm(vec_hbm, ridx_hbm, cidx_hbm, out_hbm,
          ridx_v, cidx_v, rbuf, cbuf, rows_v, zbuf, acc, sem):
    c = lax.axis_index("c")
    s = lax.axis_index("s")
    wid = s * NC + c
    base = wid * EPW
    pltpu.sync_copy(ridx_hbm.at[pl.ds(base, EPW)], ridx_v)
    pltpu.sync_copy(cidx_hbm.at[pl.ds(base, EPW)], cidx_v)

    zero = jnp.zeros((L,), jnp.float32)
    for r in range(L):
        for j in range(F // L):
            zbuf[r, pl.ds(j * L, L)] = zero

    def zacc(i, carry):
        pltpu.sync_copy(zbuf, acc.at[pl.ds(s * RPT + i * L, L)])
        return carry

    lax.fori_loop(0, RPT // L, zacc, 0)
    rem = RPT - (RPT // L) * L
    if rem:
        pltpu.sync_copy(zbuf.at[pl.ds(0, rem)],
                        acc.at[pl.ds(s * RPT + RPT - rem, rem)])

    plsc.subcore_barrier()

    def body(i, carry):
        eb = i * CH
        for j in range(CH // L):
            rbuf[pl.ds(j * L, L)] = ridx_v[pl.ds(eb + j * L, L)]
            cbuf[pl.ds(j * L, L)] = cidx_v[pl.ds(eb + j * L, L)]
        pltpu.async_copy(vec_hbm.at[cbuf], rows_v, sem).wait()
        pltpu.sync_copy(rows_v, acc.at[rbuf], add=True)
        return carry

    lax.fori_loop(0, NCHUNK, body, 0)

    plsc.subcore_barrier()
    pltpu.sync_copy(acc.at[pl.ds(s * WPT, WPT)],
                    out_hbm.at[c, pl.ds(s * WPT, WPT)])


def _dinv_body(pdeg_ref, dinv_ref):
    deg = jnp.sum(pdeg_ref[...], axis=0)
    dinv_ref[...] = jnp.where(deg > 0.0, lax.rsqrt(deg), 0.0)


_dinv_call = pl.pallas_call(
    _dinv_body,
    out_shape=jax.ShapeDtypeStruct((N,), jnp.float32),
)

BN = 2000


def _scale_body(x_ref, dv_ref, xs_ref):
    xs_ref[...] = x_ref[...] * dv_ref[...]


_scale_call = pl.pallas_call(
    _scale_body,
    grid=(N // BN,),
    in_specs=[pl.BlockSpec((BN, F), lambda i: (i, 0)),
              pl.BlockSpec((BN, 1), lambda i: (i, 0))],
    out_specs=pl.BlockSpec((BN, F), lambda i: (i, 0)),
    out_shape=jax.ShapeDtypeStruct((N, F), jnp.float32),
)


def _e1_body(x_ref, dv_ref, p0_ref, p1_ref, w0_ref, w1_ref, b_ref,
             ys_ref, po_ref):
    dv = dv_ref[...]
    tx1 = (-dv) * (p0_ref[...] + p1_ref[...])
    ys_ref[...] = dv * tx1
    po_ref[...] = (jnp.dot(x_ref[...], w0_ref[...],
                           preferred_element_type=jnp.float32)
                   + jnp.dot(tx1, w1_ref[...],
                             preferred_element_type=jnp.float32)
                   + b_ref[...])


_e1_call = pl.pallas_call(
    _e1_body,
    grid=(N // BN,),
    in_specs=[pl.BlockSpec((BN, F), lambda i: (i, 0)),
              pl.BlockSpec((BN, 1), lambda i: (i, 0)),
              pl.BlockSpec((BN, F), lambda i: (i, 0)),
              pl.BlockSpec((BN, F), lambda i: (i, 0)),
              pl.BlockSpec((F, F), lambda i: (0, 0)),
              pl.BlockSpec((F, F), lambda i: (0, 0)),
              pl.BlockSpec((1, F), lambda i: (0, 0))],
    out_specs=[pl.BlockSpec((BN, F), lambda i: (i, 0)),
               pl.BlockSpec((BN, F), lambda i: (i, 0))],
    out_shape=[jax.ShapeDtypeStruct((N, F), jnp.float32),
               jax.ShapeDtypeStruct((N, F), jnp.float32)],
)


def _e2_body(po_ref, x_ref, dv_ref, q0_ref, q1_ref, w2_ref, out_ref):
    tx2 = (-2.0 * dv_ref[...]) * (q0_ref[...] + q1_ref[...]) - x_ref[...]
    out_ref[...] = po_ref[...] + jnp.dot(tx2, w2_ref[...],
                                         preferred_element_type=jnp.float32)


_e2_call = pl.pallas_call(
    _e2_body,
    grid=(N // BN,),
    in_specs=[pl.BlockSpec((BN, F), lambda i: (i, 0)),
              pl.BlockSpec((BN, F), lambda i: (i, 0)),
              pl.BlockSpec((BN, 1), lambda i: (i, 0)),
              pl.BlockSpec((BN, F), lambda i: (i, 0)),
              pl.BlockSpec((BN, F), lambda i: (i, 0)),
              pl.BlockSpec((F, F), lambda i: (0, 0))],
    out_specs=pl.BlockSpec((BN, F), lambda i: (i, 0)),
    out_shape=jax.ShapeDtypeStruct((N, F), jnp.float32),
)


def kernel(x, edge_index, weight, bias):
    row = edge_index[0]
    col = edge_index[1]
    pdeg, ridx = _deg_ridx(row, col)
    dinv = _dinv_call(pdeg)
    dv = dinv.reshape(N, 1)
    xs = _scale_call(x, dv)
    s1 = _spmm(xs, ridx, col)
    ys, pout = _e1_call(x, dv, s1[0], s1[1], weight[0], weight[1],
                        bias.reshape(1, F))
    s2 = _spmm(ys, ridx, col)
    return _e2_call(pout, x, dv, s2[0], s2[1], weight[2])
